# Initial kernel scaffold; baseline (speedup 1.0000x reference)
import jax
import jax.numpy as jnp
from jax.experimental import pallas as pl


def kernel(x_nchw, w_pt, bias, gamma, beta):
    raise NotImplementedError("write your pallas kernel here")


# speedup vs baseline: 1.9262x; 1.9262x over previous
import jax
import jax.numpy as jnp
from jax import lax
from jax.experimental import pallas as pl
from jax.experimental.pallas import tpu as pltpu


def _make_stats_kernel(th, W, Cin, C4):
    def _body(xa_ref, xh_ref, w_ref, ssum_ref, ssq_ref):
        xa = xa_ref[0]
        xh = xh_ref[0]
        x_nxt = jnp.concatenate([xa[1:], xh], axis=0)
        x4 = jnp.concatenate(
            [xa[:, :W, :], xa[:, 1:, :], x_nxt[:, :W, :], x_nxt[:, 1:, :]],
            axis=-1)
        lhs = x4.reshape(th * W, 4 * Cin)
        yc = jnp.dot(lhs, w_ref[...], preferred_element_type=jnp.float32)
        ssum_ref[...] = jnp.sum(yc, axis=0, keepdims=True).reshape(1, 1, 1, C4)
        ssq_ref[...] = jnp.sum(yc * yc, axis=0, keepdims=True).reshape(1, 1, 1, C4)
    return _body


def _stats_pallas(xp, w_big, *, H, W, Cin, C4, th):
    N = xp.shape[0]
    n_t = H // th
    return pl.pallas_call(
        _make_stats_kernel(th, W, Cin, C4),
        out_shape=(
            jax.ShapeDtypeStruct((N, n_t, 1, C4), jnp.float32),
            jax.ShapeDtypeStruct((N, n_t, 1, C4), jnp.float32),
        ),
        grid_spec=pltpu.PrefetchScalarGridSpec(
            num_scalar_prefetch=0,
            grid=(N, n_t),
            in_specs=[
                pl.BlockSpec((1, th, W + 1, Cin), lambda n, t: (n, t, 0, 0)),
                pl.BlockSpec((1, 1, W + 1, Cin),
                             lambda n, t: (n, (t + 1) * th, 0, 0)),
                pl.BlockSpec((4 * Cin, C4), lambda n, t: (0, 0)),
            ],
            out_specs=[
                pl.BlockSpec((1, 1, 1, C4), lambda n, t: (n, t, 0, 0)),
                pl.BlockSpec((1, 1, 1, C4), lambda n, t: (n, t, 0, 0)),
            ],
        ),
        compiler_params=pltpu.CompilerParams(
            dimension_semantics=("parallel", "parallel"),
            vmem_limit_bytes=64 * 1024 * 1024),
    )(xp, xp, w_big)


def _make_fused_kernel(th, W, Cin, Cout):
    C4 = 4 * Cout

    def _body(xa_ref, xh_ref, w_ref, sh_ref, o_ref):
        xa = xa_ref[0]
        xh = xh_ref[0]
        x_nxt = jnp.concatenate([xa[1:], xh], axis=0)
        x4 = jnp.concatenate(
            [xa[:, :W, :], xa[:, 1:, :], x_nxt[:, :W, :], x_nxt[:, 1:, :]],
            axis=-1)
        rhs = x4.reshape(th * W, 4 * Cin)
        z = lax.dot_general(w_ref[...], rhs, (((0,), (1,)), ((), ())),
                            preferred_element_type=jnp.float32)
        z = jnp.maximum(z + sh_ref[...], 0.0)
        lane = lax.broadcasted_iota(jnp.int32, (Cout, 2 * W), 1)
        idx = (lane % 2) * W + lane // 2
        pieces = []
        for i in range(th):
            for a in (0, 1):
                zb0 = z[(2 * a + 0) * Cout:(2 * a + 1) * Cout,
                        i * W:(i + 1) * W]
                zb1 = z[(2 * a + 1) * Cout:(2 * a + 2) * Cout,
                        i * W:(i + 1) * W]
                d = jnp.concatenate([zb0, zb1], axis=1)
                pieces.append(jnp.take_along_axis(d, idx, axis=1))
        o_ref[0] = jnp.concatenate(pieces, axis=1)
    return _body


def _fused_pallas(xp, w_big_s, shift_col, *, H, W, Cin, Cout, th):
    N = xp.shape[0]
    C4 = 4 * Cout
    n_t = H // th
    return pl.pallas_call(
        _make_fused_kernel(th, W, Cin, Cout),
        out_shape=jax.ShapeDtypeStruct((N, Cout, 4 * H * W), jnp.float32),
        grid_spec=pltpu.PrefetchScalarGridSpec(
            num_scalar_prefetch=0,
            grid=(N, n_t),
            in_specs=[
                pl.BlockSpec((1, th, W + 1, Cin), lambda n, t: (n, t, 0, 0)),
                pl.BlockSpec((1, 1, W + 1, Cin),
                             lambda n, t: (n, (t + 1) * th, 0, 0)),
                pl.BlockSpec((4 * Cin, C4), lambda n, t: (0, 0)),
                pl.BlockSpec((C4, 1), lambda n, t: (0, 0)),
            ],
            out_specs=pl.BlockSpec((1, Cout, 4 * th * W),
                                   lambda n, t: (n, 0, t)),
        ),
        compiler_params=pltpu.CompilerParams(
            dimension_semantics=("parallel", "parallel"),
            vmem_limit_bytes=64 * 1024 * 1024),
    )(xp, xp, w_big_s, shift_col)


def kernel(x_nchw, w_pt, bias, gamma, beta, eps=1e-3):
    N, Cin, H, W = x_nchw.shape
    Cout = w_pt.shape[1]
    C4 = 4 * Cout

    x_nhwc = jnp.transpose(x_nchw, (0, 2, 3, 1))
    xp = jnp.pad(x_nhwc, ((0, 0), (0, 1), (0, 1), (0, 0))).astype(jnp.bfloat16)

    w = jnp.transpose(w_pt, (2, 3, 0, 1)).astype(jnp.float32)
    Z = jnp.zeros((Cin, Cout), jnp.float32)
    w_big = jnp.concatenate([
        jnp.concatenate([w[1, 1], w[1, 2], w[2, 1], w[2, 2]], axis=1),
        jnp.concatenate([Z,       w[1, 0], Z,       w[2, 0]], axis=1),
        jnp.concatenate([Z,       Z,       w[0, 1], w[0, 2]], axis=1),
        jnp.concatenate([Z,       Z,       Z,       w[0, 0]], axis=1),
    ], axis=0).astype(jnp.bfloat16)

    th1 = 16 if H % 16 == 0 and (16 * W) % 128 == 0 else 8
    ssum, ssq = _stats_pallas(xp, w_big, H=H, W=W, Cin=Cin, C4=C4, th=th1)

    count = jnp.float32(N * H * W * 4)
    sum_c = jnp.sum(jnp.sum(ssum, axis=(0, 1, 2)).reshape(4, Cout), axis=0)
    ssq_c = jnp.sum(jnp.sum(ssq, axis=(0, 1, 2)).reshape(4, Cout), axis=0)
    mean_c = sum_c / count
    var_c = jnp.maximum(ssq_c / count - mean_c * mean_c, 0.0)
    scale = gamma * lax.rsqrt(var_c + eps)
    shift = beta - mean_c * scale
    scale_row4 = jnp.tile(scale, 4).reshape(1, C4)
    w_big_s = (w_big.astype(jnp.float32) * scale_row4).astype(jnp.bfloat16)
    shift_col = jnp.tile(shift, 4).reshape(C4, 1)

    th2 = 8 if H % 8 == 0 and (8 * W) % 128 == 0 else H
    out_flat = _fused_pallas(xp, w_big_s, shift_col,
                             H=H, W=W, Cin=Cin, Cout=Cout, th=th2)
    return out_flat.reshape(N, Cout, 2 * H, 2 * W)


# speedup vs baseline: 2.0990x; 1.0897x over previous
import jax
import jax.numpy as jnp
from jax import lax
from jax.experimental import pallas as pl
from jax.experimental.pallas import tpu as pltpu


def _make_stats_kernel(th, W, Cin, C4):
    def _body(xa_ref, xh_ref, w_ref, ssum_ref, ssq_ref):
        xa = xa_ref[0]
        xh = xh_ref[0]
        x_nxt = jnp.concatenate([xa[1:], xh], axis=0)
        x4 = jnp.concatenate(
            [xa[:, :W, :], xa[:, 1:, :], x_nxt[:, :W, :], x_nxt[:, 1:, :]],
            axis=-1)
        lhs = x4.reshape(th * W, 4 * Cin)
        yc = jnp.dot(lhs, w_ref[...], preferred_element_type=jnp.float32)
        ssum_ref[...] = jnp.sum(yc, axis=0, keepdims=True).reshape(1, 1, 1, C4)
        ssq_ref[...] = jnp.sum(yc * yc, axis=0, keepdims=True).reshape(1, 1, 1, C4)
    return _body


def _stats_pallas(xp, w_big, *, H, W, Cin, C4, th):
    N = xp.shape[0]
    n_t = H // th
    return pl.pallas_call(
        _make_stats_kernel(th, W, Cin, C4),
        out_shape=(
            jax.ShapeDtypeStruct((N, n_t, 1, C4), jnp.float32),
            jax.ShapeDtypeStruct((N, n_t, 1, C4), jnp.float32),
        ),
        grid_spec=pltpu.PrefetchScalarGridSpec(
            num_scalar_prefetch=0,
            grid=(N, n_t),
            in_specs=[
                pl.BlockSpec((1, th, W + 1, Cin), lambda n, t: (n, t, 0, 0)),
                pl.BlockSpec((1, 1, W + 1, Cin),
                             lambda n, t: (n, (t + 1) * th, 0, 0)),
                pl.BlockSpec((4 * Cin, C4), lambda n, t: (0, 0)),
            ],
            out_specs=[
                pl.BlockSpec((1, 1, 1, C4), lambda n, t: (n, t, 0, 0)),
                pl.BlockSpec((1, 1, 1, C4), lambda n, t: (n, t, 0, 0)),
            ],
        ),
        compiler_params=pltpu.CompilerParams(
            dimension_semantics=("parallel", "parallel"),
            vmem_limit_bytes=64 * 1024 * 1024),
    )(xp, xp, w_big)


def _make_fused_kernel(th, W, Cin, Cout):
    C4 = 4 * Cout

    def _body(xa_ref, xh_ref, w_ref, sh_ref, o_ref):
        xa = xa_ref[0]
        xh = xh_ref[0]
        x_nxt = jnp.concatenate([xa[1:], xh], axis=0)
        x4 = jnp.concatenate(
            [xa[:, :W, :], xa[:, 1:, :], x_nxt[:, :W, :], x_nxt[:, 1:, :]],
            axis=-1)
        rhs = x4.reshape(th * W, 4 * Cin)
        z = lax.dot_general(w_ref[...], rhs, (((0,), (1,)), ((), ())),
                            preferred_element_type=jnp.float32)
        sh = sh_ref[...]
        lane = lax.broadcasted_iota(jnp.int32, (Cout, 2 * W), 1)
        idx = (lane % 2) * W + lane // 2
        pieces = []
        for i in range(th):
            for a in (0, 1):
                zb0 = z[(2 * a + 0) * Cout:(2 * a + 1) * Cout,
                        i * W:(i + 1) * W]
                zb1 = z[(2 * a + 1) * Cout:(2 * a + 2) * Cout,
                        i * W:(i + 1) * W]
                d = jnp.concatenate([zb0, zb1], axis=1)
                pieces.append(jnp.maximum(
                    jnp.take_along_axis(d, idx, axis=1) + sh, 0.0))
        o_ref[0] = jnp.concatenate(pieces, axis=1)
    return _body


def _fused_pallas(xp, w_big_s, shift_bc, *, H, W, Cin, Cout, th):
    N = xp.shape[0]
    C4 = 4 * Cout
    n_t = H // th
    return pl.pallas_call(
        _make_fused_kernel(th, W, Cin, Cout),
        out_shape=jax.ShapeDtypeStruct((N, Cout, 4 * H * W), jnp.float32),
        grid_spec=pltpu.PrefetchScalarGridSpec(
            num_scalar_prefetch=0,
            grid=(N, n_t),
            in_specs=[
                pl.BlockSpec((1, th, W + 1, Cin), lambda n, t: (n, t, 0, 0)),
                pl.BlockSpec((1, 1, W + 1, Cin),
                             lambda n, t: (n, (t + 1) * th, 0, 0)),
                pl.BlockSpec((4 * Cin, C4), lambda n, t: (0, 0)),
                pl.BlockSpec((Cout, 2 * W), lambda n, t: (0, 0)),
            ],
            out_specs=pl.BlockSpec((1, Cout, 4 * th * W),
                                   lambda n, t: (n, 0, t)),
        ),
        compiler_params=pltpu.CompilerParams(
            dimension_semantics=("parallel", "parallel"),
            vmem_limit_bytes=64 * 1024 * 1024),
    )(xp, xp, w_big_s, shift_bc)


def kernel(x_nchw, w_pt, bias, gamma, beta, eps=1e-3):
    N, Cin, H, W = x_nchw.shape
    Cout = w_pt.shape[1]
    C4 = 4 * Cout

    x_nhwc = jnp.transpose(x_nchw, (0, 2, 3, 1))
    xp = jnp.pad(x_nhwc, ((0, 0), (0, 1), (0, 1), (0, 0))).astype(jnp.bfloat16)

    w = jnp.transpose(w_pt, (2, 3, 0, 1)).astype(jnp.float32)
    Z = jnp.zeros((Cin, Cout), jnp.float32)
    w_big = jnp.concatenate([
        jnp.concatenate([w[1, 1], w[1, 2], w[2, 1], w[2, 2]], axis=1),
        jnp.concatenate([Z,       w[1, 0], Z,       w[2, 0]], axis=1),
        jnp.concatenate([Z,       Z,       w[0, 1], w[0, 2]], axis=1),
        jnp.concatenate([Z,       Z,       Z,       w[0, 0]], axis=1),
    ], axis=0).astype(jnp.bfloat16)

    th1 = H if (H * W) % 128 == 0 else 8
    ssum, ssq = _stats_pallas(xp, w_big, H=H, W=W, Cin=Cin, C4=C4, th=th1)

    count = jnp.float32(N * H * W * 4)
    sum_c = jnp.sum(jnp.sum(ssum, axis=(0, 1, 2)).reshape(4, Cout), axis=0)
    ssq_c = jnp.sum(jnp.sum(ssq, axis=(0, 1, 2)).reshape(4, Cout), axis=0)
    mean_c = sum_c / count
    var_c = jnp.maximum(ssq_c / count - mean_c * mean_c, 0.0)
    scale = gamma * lax.rsqrt(var_c + eps)
    shift = beta - mean_c * scale
    scale_row4 = jnp.tile(scale, 4).reshape(1, C4)
    w_big_s = (w_big.astype(jnp.float32) * scale_row4).astype(jnp.bfloat16)
    shift_bc = jnp.broadcast_to(shift.reshape(Cout, 1), (Cout, 2 * W))

    th2 = 8 if H % 8 == 0 and (8 * W) % 128 == 0 else H
    out_flat = _fused_pallas(xp, w_big_s, shift_bc,
                             H=H, W=W, Cin=Cin, Cout=Cout, th=th2)
    return out_flat.reshape(N, Cout, 2 * H, 2 * W)


# speedup vs baseline: 2.1053x; 1.0030x over previous
import jax
import jax.numpy as jnp
from jax import lax
from jax.experimental import pallas as pl
from jax.experimental.pallas import tpu as pltpu


def _make_stats_kernel(th, W, Cin, C4):
    def _body(xa_ref, xh_ref, w_ref, ssum_ref, ssq_ref):
        xa = xa_ref[0]
        xh = xh_ref[0]
        x_nxt = jnp.concatenate([xa[1:], xh], axis=0)
        x4 = jnp.concatenate(
            [xa[:, :W, :], xa[:, 1:, :], x_nxt[:, :W, :], x_nxt[:, 1:, :]],
            axis=-1)
        lhs = x4.reshape(th * W, 4 * Cin)
        yc = jnp.dot(lhs, w_ref[...], preferred_element_type=jnp.float32)
        ssum_ref[...] = jnp.sum(yc, axis=0, keepdims=True).reshape(1, 1, 1, C4)
        ssq_ref[...] = jnp.sum(yc * yc, axis=0, keepdims=True).reshape(1, 1, 1, C4)
    return _body


def _stats_pallas(xp, w_big, *, H, W, Cin, C4, th):
    N = xp.shape[0]
    n_t = H // th
    return pl.pallas_call(
        _make_stats_kernel(th, W, Cin, C4),
        out_shape=(
            jax.ShapeDtypeStruct((N, n_t, 1, C4), jnp.float32),
            jax.ShapeDtypeStruct((N, n_t, 1, C4), jnp.float32),
        ),
        grid_spec=pltpu.PrefetchScalarGridSpec(
            num_scalar_prefetch=0,
            grid=(N, n_t),
            in_specs=[
                pl.BlockSpec((1, th, W + 1, Cin), lambda n, t: (n, t, 0, 0)),
                pl.BlockSpec((1, 1, W + 1, Cin),
                             lambda n, t: (n, (t + 1) * th, 0, 0)),
                pl.BlockSpec((4 * Cin, C4), lambda n, t: (0, 0)),
            ],
            out_specs=[
                pl.BlockSpec((1, 1, 1, C4), lambda n, t: (n, t, 0, 0)),
                pl.BlockSpec((1, 1, 1, C4), lambda n, t: (n, t, 0, 0)),
            ],
        ),
        compiler_params=pltpu.CompilerParams(
            dimension_semantics=("parallel", "parallel"),
            vmem_limit_bytes=64 * 1024 * 1024),
    )(xp, xp, w_big)


def _make_fused_kernel(th, W, Cin, Cout):
    C4 = 4 * Cout

    def _body(xa_ref, xh_ref, w_ref, sh_ref, o_ref):
        xa = xa_ref[0]
        xh = xh_ref[0]
        x_nxt = jnp.concatenate([xa[1:], xh], axis=0)
        x4 = jnp.concatenate(
            [xa[:, :W, :], xa[:, 1:, :], x_nxt[:, :W, :], x_nxt[:, 1:, :]],
            axis=-1)
        rhs = x4.reshape(th * W, 4 * Cin)
        z = lax.dot_general(w_ref[...], rhs, (((0,), (1,)), ((), ())),
                            preferred_element_type=jnp.float32)
        sh = sh_ref[...]
        lane = lax.broadcasted_iota(jnp.int32, (Cout, 2 * W), 1)
        idx = (lane % 2) * W + lane // 2
        for i in range(th):
            for a in (0, 1):
                zb0 = z[(2 * a + 0) * Cout:(2 * a + 1) * Cout,
                        i * W:(i + 1) * W]
                zb1 = z[(2 * a + 1) * Cout:(2 * a + 2) * Cout,
                        i * W:(i + 1) * W]
                d = jnp.concatenate([zb0, zb1], axis=1)
                piece = jnp.maximum(
                    jnp.take_along_axis(d, idx, axis=1) + sh, 0.0)
                k = 2 * i + a
                o_ref[0, :, k * 2 * W:(k + 1) * 2 * W] = piece
    return _body


def _fused_pallas(xp, w_big_s, shift_bc, *, H, W, Cin, Cout, th):
    N = xp.shape[0]
    C4 = 4 * Cout
    n_t = H // th
    return pl.pallas_call(
        _make_fused_kernel(th, W, Cin, Cout),
        out_shape=jax.ShapeDtypeStruct((N, Cout, 4 * H * W), jnp.float32),
        grid_spec=pltpu.PrefetchScalarGridSpec(
            num_scalar_prefetch=0,
            grid=(N, n_t),
            in_specs=[
                pl.BlockSpec((1, th, W + 1, Cin), lambda n, t: (n, t, 0, 0)),
                pl.BlockSpec((1, 1, W + 1, Cin),
                             lambda n, t: (n, (t + 1) * th, 0, 0)),
                pl.BlockSpec((4 * Cin, C4), lambda n, t: (0, 0)),
                pl.BlockSpec((Cout, 2 * W), lambda n, t: (0, 0)),
            ],
            out_specs=pl.BlockSpec((1, Cout, 4 * th * W),
                                   lambda n, t: (n, 0, t)),
        ),
        compiler_params=pltpu.CompilerParams(
            dimension_semantics=("parallel", "parallel"),
            vmem_limit_bytes=64 * 1024 * 1024),
    )(xp, xp, w_big_s, shift_bc)


def kernel(x_nchw, w_pt, bias, gamma, beta, eps=1e-3):
    N, Cin, H, W = x_nchw.shape
    Cout = w_pt.shape[1]
    C4 = 4 * Cout

    x_nhwc = jnp.transpose(x_nchw, (0, 2, 3, 1))
    xp = jnp.pad(x_nhwc, ((0, 0), (0, 1), (0, 1), (0, 0))).astype(jnp.bfloat16)

    w = jnp.transpose(w_pt, (2, 3, 0, 1)).astype(jnp.float32)
    Z = jnp.zeros((Cin, Cout), jnp.float32)
    w_big = jnp.concatenate([
        jnp.concatenate([w[1, 1], w[1, 2], w[2, 1], w[2, 2]], axis=1),
        jnp.concatenate([Z,       w[1, 0], Z,       w[2, 0]], axis=1),
        jnp.concatenate([Z,       Z,       w[0, 1], w[0, 2]], axis=1),
        jnp.concatenate([Z,       Z,       Z,       w[0, 0]], axis=1),
    ], axis=0)
    w_bf = w_big.astype(jnp.bfloat16)

    th1 = H if (H * W) % 128 == 0 else 8
    ssum, ssq = _stats_pallas(xp, w_bf, H=H, W=W, Cin=Cin, C4=C4, th=th1)

    count = jnp.float32(N * H * W * 4)
    sum_c = jnp.sum(jnp.sum(ssum, axis=(0, 1, 2)).reshape(4, Cout), axis=0)
    ssq_c = jnp.sum(jnp.sum(ssq, axis=(0, 1, 2)).reshape(4, Cout), axis=0)
    mean_c = sum_c / count
    var_c = jnp.maximum(ssq_c / count - mean_c * mean_c, 0.0)
    scale = gamma * lax.rsqrt(var_c + eps)
    shift = beta - mean_c * scale
    scale_row4 = jnp.tile(scale, 4).reshape(1, C4)
    w_big_s = (w_big * scale_row4).astype(jnp.bfloat16)
    shift_bc = jnp.broadcast_to(shift.reshape(Cout, 1), (Cout, 2 * W))

    th2 = 16 if (H % 16 == 0 and (16 * W) % 128 == 0) else \
        (8 if (H % 8 == 0 and (8 * W) % 128 == 0) else H)
    out_flat = _fused_pallas(xp, w_big_s, shift_bc,
                             H=H, W=W, Cin=Cin, Cout=Cout, th=th2)
    return out_flat.reshape(N, Cout, 2 * H, 2 * W)
